# speedup vs baseline: 4.4919x; 4.4919x over previous
import functools
import math

import jax
import jax.numpy as jnp
from jax import lax
from jax.experimental import pallas as pl
from jax.experimental.pallas import tpu as pltpu
from jax.experimental.pallas import tpu_sc as plsc

D_MODEL = 128
SCALE = math.sqrt(D_MODEL)

_NC = 2
_NS = 16
_NW = _NC * _NS
_LANES = 16

_CH = 128


@functools.lru_cache(maxsize=None)
def _make_kernel(B: int):
    assert B % (_NW * _CH) == 0
    n_per_w = B // _NW
    n_chunks = n_per_w // _CH
    mesh = plsc.VectorSubcoreMesh(core_axis_name="c", subcore_axis_name="s")

    @functools.partial(
        pl.kernel,
        mesh=mesh,
        out_type=jax.ShapeDtypeStruct((B, D_MODEL), jnp.float32),
        scratch_types=[
            pltpu.VMEM((_CH,), jnp.int32),
            pltpu.VMEM((_CH, D_MODEL), jnp.float32),
            pltpu.SemaphoreType.DMA,
        ],
    )
    def gather_scale(x_hbm, table_hbm, out_hbm, idx_v, rows_v, sem):
        wid = lax.axis_index("s") * _NC + lax.axis_index("c")
        base = wid * n_per_w

        def chunk_body(g, carry):
            off = base + g * _CH
            pltpu.sync_copy(x_hbm.at[pl.ds(off, _CH)], idx_v)
            pltpu.async_copy(table_hbm.at[idx_v], rows_v, sem).wait()

            def row_body(i, c):
                for j in range(D_MODEL // _LANES):
                    sl = pl.ds(j * _LANES, _LANES)
                    rows_v[i, sl] = rows_v[i, sl] * SCALE
                return c

            lax.fori_loop(0, _CH, row_body, 0, unroll=False)
            pltpu.sync_copy(rows_v, out_hbm.at[pl.ds(off, _CH)])
            return carry

        lax.fori_loop(0, n_chunks, chunk_body, 0, unroll=False)

    return gather_scale


def kernel(x, table):
    S, T = x.shape
    B = S * T
    x_flat = x.reshape(B).astype(jnp.int32)
    out = _make_kernel(B)(x_flat, table)
    return out.reshape(S, T, D_MODEL)


# speedup vs baseline: 9.2444x; 2.0580x over previous
import functools
import math

import jax
import jax.numpy as jnp
from jax import lax
from jax.experimental import pallas as pl
from jax.experimental.pallas import tpu as pltpu
from jax.experimental.pallas import tpu_sc as plsc

D_MODEL = 128
SCALE = math.sqrt(D_MODEL)

_NC = 2
_NS = 16
_NW = _NC * _NS
_LANES = 16

_CH = 128
_NBUF = 2


@functools.lru_cache(maxsize=None)
def _make_kernel(B: int):
    assert B % (_NW * _CH * _NBUF) == 0
    n_per_w = B // _NW
    n_chunks = n_per_w // _CH
    n_trips = n_chunks // _NBUF
    mesh = plsc.VectorSubcoreMesh(core_axis_name="c", subcore_axis_name="s")

    @functools.partial(
        pl.kernel,
        mesh=mesh,
        out_type=jax.ShapeDtypeStruct((B, D_MODEL), jnp.float32),
        scratch_types=[
            pltpu.VMEM((n_per_w,), jnp.int32),
            pltpu.VMEM((_NBUF, _CH, D_MODEL), jnp.float32),
            pltpu.VMEM((_NBUF, _CH, D_MODEL), jnp.float32),
            pltpu.SemaphoreType.DMA((_NBUF,)),
            pltpu.SemaphoreType.DMA((_NBUF,)),
        ],
    )
    def gather_scale(x_hbm, table_hbm, out_hbm, idx_all, rows_in, rows_out,
                     gsem, osem):
        wid = lax.axis_index("s") * _NC + lax.axis_index("c")
        base = wid * n_per_w

        pltpu.sync_copy(x_hbm.at[pl.ds(base, n_per_w)], idx_all)

        def fire_gather(g, b):
            pltpu.async_copy(
                table_hbm.at[idx_all.at[pl.ds(g * _CH, _CH)]],
                rows_in.at[b], gsem.at[b])

        for b in range(_NBUF):
            fire_gather(b, b)

        def trip_body(t, carry):
            for b in range(_NBUF):
                g = t * _NBUF + b
                off = base + g * _CH
                pltpu.make_async_copy(
                    table_hbm.at[idx_all.at[pl.ds(g * _CH, _CH)]],
                    rows_in.at[b], gsem.at[b]).wait()

                @pl.when(t > 0)
                def _wait_out():
                    pltpu.make_async_copy(
                        rows_out.at[b], out_hbm.at[pl.ds(base, _CH)],
                        osem.at[b]).wait()

                rin = rows_in.at[b]
                rout = rows_out.at[b]

                def row_body(i, c):
                    for j in range(D_MODEL // _LANES):
                        sl = pl.ds(j * _LANES, _LANES)
                        rout[i, sl] = rin[i, sl] * SCALE
                    return c

                lax.fori_loop(0, _CH, row_body, 0, unroll=False)
                pltpu.async_copy(rout, out_hbm.at[pl.ds(off, _CH)],
                                 osem.at[b])

                @pl.when(t < n_trips - 1)
                def _next_gather():
                    fire_gather(g + _NBUF, b)
            return carry

        lax.fori_loop(0, n_trips, trip_body, 0, unroll=False)

        for b in range(_NBUF):
            pltpu.make_async_copy(
                rows_out.at[b], out_hbm.at[pl.ds(base, _CH)],
                osem.at[b]).wait()

    return gather_scale


def kernel(x, table):
    S, T = x.shape
    B = S * T
    x_flat = x.reshape(B).astype(jnp.int32)
    out = _make_kernel(B)(x_flat, table)
    return out.reshape(S, T, D_MODEL)
